# baseline (speedup 1.0000x reference)
import functools
import math

import jax
import jax.numpy as jnp
from jax import lax
from jax.experimental import pallas as pl
from jax.experimental.pallas import tpu as pltpu
from jax.experimental.pallas import tpu_sc as plsc

_PADDING_IDX = 0
_SMOOTHING = 0.1
_CONFIDENCE = 1.0 - _SMOOTHING

_B, _T, _C = 4, 2048, 4096
_N = _B * _T

_NC, _NS, _L = 2, 16, 16
_NW = _NC * _NS
_ROWS_PER_W = _N // _NW
_LCHUNKS = _ROWS_PER_W // _L

_TC_BLK = 256
_TC_GRID = _N // _TC_BLK

_S = _SMOOTHING / (_C - 2)
_K = (_C - 2) * _S * math.log(_S) + _CONFIDENCE * math.log(_CONFIDENCE)


def _sc_body(x_hbm, lab_hbm, out_hbm, labv, idxv, gathv, outv, sem):
    wid = lax.axis_index("s") * _NC + lax.axis_index("c")
    base = wid * _ROWS_PER_W

    pltpu.sync_copy(lab_hbm.at[pl.ds(base, _ROWS_PER_W)], labv)

    lane = lax.iota(jnp.int32, _L)
    for ci in range(_LCHUNKS):
        lab16 = labv[pl.ds(ci * _L, _L)]
        idxv[pl.ds(ci * _L, _L)] = (base + ci * _L + lane) * _C + lab16

    pltpu.async_copy(x_hbm.at[idxv], gathv, sem).wait()

    acc = jnp.zeros((_L,), jnp.float32)
    for ci in range(_LCHUNKS):
        lab16 = labv[pl.ds(ci * _L, _L)]
        v = gathv[pl.ds(ci * _L, _L)]
        acc = acc + jnp.where(lab16 != _PADDING_IDX, v, jnp.float32(0.0))

    outv[...] = acc
    pltpu.sync_copy(outv, out_hbm.at[wid])


def _tc_body(lab_ref, x_ref, out_ref):
    i = pl.program_id(0)
    xb = x_ref[...]
    lab = lab_ref[0, 0, :]
    m = lab != _PADDING_IDX
    rowsum = jnp.sum(xb, axis=1)
    x0 = xb[:, 0]
    part = jnp.sum(jnp.where(m, rowsum - x0, jnp.float32(0.0)))
    cnt = jnp.sum(m.astype(jnp.float32))

    @pl.when(i == 0)
    def _():
        out_ref[0, 0] = jnp.float32(0.0)
        out_ref[0, 1] = jnp.float32(0.0)

    out_ref[0, 0] += part
    out_ref[0, 1] += cnt


@jax.jit
def _loss(x2d, x1d, lab1d, lab3d):
    mesh = plsc.VectorSubcoreMesh(core_axis_name="c", subcore_axis_name="s")
    sc_run = functools.partial(
        pl.kernel,
        out_type=jax.ShapeDtypeStruct((_NW, _L), jnp.float32),
        mesh=mesh,
        scratch_types=[
            pltpu.VMEM((_ROWS_PER_W,), jnp.int32),
            pltpu.VMEM((_ROWS_PER_W,), jnp.int32),
            pltpu.VMEM((_ROWS_PER_W,), jnp.float32),
            pltpu.VMEM((_L,), jnp.float32),
            pltpu.SemaphoreType.DMA,
        ],
        compiler_params=pltpu.CompilerParams(use_tc_tiling_on_sc=False,
                                             needs_layout_passes=False),
    )(_sc_body)
    sc_partials = sc_run(x1d, lab1d)

    tc_out = pl.pallas_call(
        _tc_body,
        grid=(_TC_GRID,),
        in_specs=[
            pl.BlockSpec((1, 1, _TC_BLK), lambda i: (i, 0, 0)),
            pl.BlockSpec((_TC_BLK, _C), lambda i: (i, 0)),
        ],
        out_specs=pl.BlockSpec(memory_space=pltpu.SMEM),
        out_shape=jax.ShapeDtypeStruct((1, 2), jnp.float32),
        compiler_params=pltpu.CompilerParams(
            dimension_semantics=("arbitrary",)),
    )(lab3d, x2d)

    xlab_sum = jnp.sum(sc_partials)
    part = tc_out[0, 0]
    cnt = tc_out[0, 1]
    return (cnt * jnp.float32(_K) - jnp.float32(_S) * part
            - jnp.float32(_CONFIDENCE - _S) * xlab_sum)


def kernel(inputs, input_sizes, labels, label_sizes):
    x2d = inputs.reshape(_N, _C)
    x1d = inputs.reshape(_N * _C)
    lab1d = labels.reshape(_N).astype(jnp.int32)
    lab3d = lab1d.reshape(_TC_GRID, 1, _TC_BLK)
    return _loss(x2d, x1d, lab1d, lab3d)


# speedup vs baseline: 2.3905x; 2.3905x over previous
import functools
import math

import jax
import jax.numpy as jnp
from jax import lax
from jax.experimental import pallas as pl
from jax.experimental.pallas import tpu as pltpu
from jax.experimental.pallas import tpu_sc as plsc

_PADDING_IDX = 0
_SMOOTHING = 0.1
_CONFIDENCE = 1.0 - _SMOOTHING

_B, _T, _C = 4, 2048, 4096
_N = _B * _T

_NC, _NS, _L = 2, 16, 16
_NW = _NC * _NS
_ROWS_PER_W = _N // _NW
_CHUNK = 8
_NCHUNK = _ROWS_PER_W // _CHUNK
_NSUPER = _NCHUNK // 2
_NBUF = 3
_UNROLL = 16
_ROW_ITERS = _C // _L // _UNROLL

_S = _SMOOTHING / (_C - 2)
_K = (_C - 2) * _S * math.log(_S) + _CONFIDENCE * math.log(_CONFIDENCE)


def _body(x_hbm, lab_hbm, out_hbm, buf, labv, outv, sem):
    wid = lax.axis_index("s") * _NC + lax.axis_index("c")
    base = wid * _ROWS_PER_W

    pltpu.sync_copy(lab_hbm.at[pl.ds(pl.multiple_of(base, _L),
                                     _ROWS_PER_W)], labv)

    s = jnp.float32(_S)
    coef = jnp.float32(_CONFIDENCE - _S)
    kconst = jnp.float32(_K)
    zero = jnp.zeros((_L,), jnp.float32)
    fzero = jnp.float32(0.0)
    lane = lax.iota(jnp.int32, _L)

    def issue(chunk):
        pltpu.async_copy(
            x_hbm.at[pl.ds(pl.multiple_of(base + chunk * _CHUNK, _CHUNK),
                           _CHUNK)],
            buf.at[lax.rem(chunk, _NBUF)], sem)

    def wait(chunk):
        pltpu.make_async_copy(
            x_hbm.at[pl.ds(0, _CHUNK)],
            buf.at[lax.rem(chunk, _NBUF)], sem).wait()

    def process(parity, lab16, lane_base, acc_sv, acc_rs):
        accs = (zero, zero, zero, zero)
        for r in range(_CHUNK):
            def col_body(j, vaccs, r=r):
                a0, a1, a2, a3 = vaccs
                aa = [a0, a1, a2, a3]
                for u in range(_UNROLL):
                    off = pl.multiple_of((j * _UNROLL + u) * _L, _L)
                    aa[u % 4] = aa[u % 4] + buf[parity, r, pl.ds(off, _L)]
                return tuple(aa)

            accs = lax.fori_loop(0, _ROW_ITERS, col_body, accs)
        a0, a1, a2, a3 = accs
        acc_rs = acc_rs + ((a0 + a1) + (a2 + a3))

        for r in range(_CHUNK):
            lab_r = lab16[lane_base + r]
            m_bool = lab_r != _PADDING_IDX
            off = pl.multiple_of((lab_r >> 4) << 4, _L)
            vl = buf[parity, r, pl.ds(off, _L)]
            v0 = buf[parity, r, pl.ds(0, _L)]
            m_f = jnp.where(m_bool, jnp.float32(1.0), fzero)
            t0 = m_f * (kconst + s * v0[0])
            selmask = jnp.logical_and(lane == (lab_r & (_L - 1)), m_bool)
            acc_sv = (acc_sv + jnp.where(lane == 0, t0, fzero)
                      - jnp.where(selmask, coef * vl, fzero))

        sub16 = lax.iota(jnp.int32, _L) - lane_base
        inv16 = jnp.logical_and(
            jnp.logical_and(sub16 >= 0, sub16 < _CHUNK),
            lab16 == _PADDING_IDX)
        ninv = jnp.sum(jnp.where(inv16, 1, 0))

        def inv_all():
            corr = zero
            for r in range(_CHUNK):
                def inv_body(j, vacc, r=r):
                    for u in range(8):
                        off = pl.multiple_of((j * 8 + u) * _L, _L)
                        vacc = vacc + buf[parity, r, pl.ds(off, _L)]
                    return vacc

                corr = corr + lax.cond(
                    lab16[lane_base + r] == _PADDING_IDX,
                    lambda b=inv_body: lax.fori_loop(
                        0, _C // (8 * _L), b, zero),
                    lambda: zero)
            return corr

        corr = lax.cond(ninv > 0, inv_all, lambda: zero)
        return acc_sv, acc_rs - corr

    issue(0)
    issue(1)

    def super_body(ci2, carry):
        acc_sv, acc_rs = carry
        c0 = 2 * ci2
        lab16 = labv[pl.ds(pl.multiple_of(ci2 * _L, _L), _L)]

        @pl.when(c0 + 2 < _NCHUNK)
        def _():
            issue(c0 + 2)

        wait(c0)
        acc_sv, acc_rs = process(lax.rem(c0, _NBUF), lab16, 0,
                                 acc_sv, acc_rs)

        @pl.when(c0 + 3 < _NCHUNK)
        def _():
            issue(c0 + 3)

        wait(c0 + 1)
        acc_sv, acc_rs = process(lax.rem(c0 + 1, _NBUF), lab16, _CHUNK,
                                 acc_sv, acc_rs)
        return acc_sv, acc_rs

    acc_sv, acc_rs = lax.fori_loop(0, _NSUPER, super_body, (zero, zero))

    outv[...] = acc_sv - s * acc_rs
    pltpu.sync_copy(outv,
                    out_hbm.at[pl.ds(pl.multiple_of(wid * _L, _L), _L)])


@jax.jit
def _loss(x2d, lab1d):
    mesh = plsc.VectorSubcoreMesh(core_axis_name="c", subcore_axis_name="s")
    run = functools.partial(
        pl.kernel,
        out_type=jax.ShapeDtypeStruct((_NW * _L,), jnp.float32),
        mesh=mesh,
        scratch_types=[
            pltpu.VMEM((_NBUF, _CHUNK, _C), jnp.float32),
            pltpu.VMEM((_ROWS_PER_W,), jnp.int32),
            pltpu.VMEM((_L,), jnp.float32),
            pltpu.SemaphoreType.DMA,
        ],
        compiler_params=pltpu.CompilerParams(use_tc_tiling_on_sc=True,
                                             needs_layout_passes=False),
    )(_body)
    partials = run(x2d, lab1d)
    return jnp.sum(partials)


def kernel(inputs, input_sizes, labels, label_sizes):
    x2d = inputs.reshape(_N, _C)
    lab1d = labels.reshape(_N).astype(jnp.int32)
    return _loss(x2d, lab1d)


# speedup vs baseline: 2.4403x; 1.0208x over previous
import functools
import math

import jax
import jax.numpy as jnp
from jax import lax
from jax.experimental import pallas as pl
from jax.experimental.pallas import tpu as pltpu
from jax.experimental.pallas import tpu_sc as plsc

_PADDING_IDX = 0
_SMOOTHING = 0.1
_CONFIDENCE = 1.0 - _SMOOTHING

_B, _T, _C = 4, 2048, 4096
_N = _B * _T

_NC, _NS, _L = 2, 16, 16
_NW = _NC * _NS

_NT = 4096
_NSC = _N - _NT
_TC_BLK = 256
_TC_GRID = _NT // _TC_BLK

_ROWS_PER_W = _NSC // _NW
_CHUNK = 8
_NCHUNK = _ROWS_PER_W // _CHUNK
_NSUPER = _NCHUNK // 2
_NBUF = 3
_UNROLL = 16
_ROW_ITERS = _C // _L // _UNROLL

_S = _SMOOTHING / (_C - 2)
_K = (_C - 2) * _S * math.log(_S) + _CONFIDENCE * math.log(_CONFIDENCE)


def _body(x_hbm, lab_hbm, out_hbm, buf, labv, outv, sem):
    wid = lax.axis_index("s") * _NC + lax.axis_index("c")
    base = _NT + wid * _ROWS_PER_W

    pltpu.sync_copy(lab_hbm.at[pl.ds(pl.multiple_of(base, _L),
                                     _ROWS_PER_W)], labv)

    s = jnp.float32(_S)
    coef = jnp.float32(_CONFIDENCE - _S)
    kconst = jnp.float32(_K)
    zero = jnp.zeros((_L,), jnp.float32)
    fzero = jnp.float32(0.0)
    lane = lax.iota(jnp.int32, _L)

    def issue(chunk):
        pltpu.async_copy(
            x_hbm.at[pl.ds(pl.multiple_of(base + chunk * _CHUNK, _CHUNK),
                           _CHUNK)],
            buf.at[lax.rem(chunk, _NBUF)], sem)

    def wait(chunk):
        pltpu.make_async_copy(
            x_hbm.at[pl.ds(0, _CHUNK)],
            buf.at[lax.rem(chunk, _NBUF)], sem).wait()

    def process(parity, lab16, lane_base, acc_sv, acc_rs):
        accs = (zero, zero, zero, zero)
        for r in range(_CHUNK):
            def col_body(j, vaccs, r=r):
                a0, a1, a2, a3 = vaccs
                aa = [a0, a1, a2, a3]
                for u in range(_UNROLL):
                    off = pl.multiple_of((j * _UNROLL + u) * _L, _L)
                    aa[u % 4] = aa[u % 4] + buf[parity, r, pl.ds(off, _L)]
                return tuple(aa)

            accs = lax.fori_loop(0, _ROW_ITERS, col_body, accs)
        a0, a1, a2, a3 = accs
        acc_rs = acc_rs + ((a0 + a1) + (a2 + a3))

        for r in range(_CHUNK):
            lab_r = lab16[lane_base + r]
            m_bool = lab_r != _PADDING_IDX
            off = pl.multiple_of((lab_r >> 4) << 4, _L)
            vl = buf[parity, r, pl.ds(off, _L)]
            v0 = buf[parity, r, pl.ds(0, _L)]
            m_f = jnp.where(m_bool, jnp.float32(1.0), fzero)
            t0 = m_f * (kconst + s * v0[0])
            selmask = jnp.logical_and(lane == (lab_r & (_L - 1)), m_bool)
            acc_sv = (acc_sv + jnp.where(lane == 0, t0, fzero)
                      - jnp.where(selmask, coef * vl, fzero))

        sub16 = lax.iota(jnp.int32, _L) - lane_base
        inv16 = jnp.logical_and(
            jnp.logical_and(sub16 >= 0, sub16 < _CHUNK),
            lab16 == _PADDING_IDX)
        ninv = jnp.sum(jnp.where(inv16, 1, 0))

        def inv_all():
            corr = zero
            for r in range(_CHUNK):
                def inv_body(j, vacc, r=r):
                    for u in range(8):
                        off = pl.multiple_of((j * 8 + u) * _L, _L)
                        vacc = vacc + buf[parity, r, pl.ds(off, _L)]
                    return vacc

                corr = corr + lax.cond(
                    lab16[lane_base + r] == _PADDING_IDX,
                    lambda b=inv_body: lax.fori_loop(
                        0, _C // (8 * _L), b, zero),
                    lambda: zero)
            return corr

        corr = lax.cond(ninv > 0, inv_all, lambda: zero)
        return acc_sv, acc_rs - corr

    issue(0)
    issue(1)

    def super_body(ci2, carry):
        acc_sv, acc_rs = carry
        c0 = 2 * ci2
        lab16 = labv[pl.ds(pl.multiple_of(ci2 * _L, _L), _L)]

        @pl.when(c0 + 2 < _NCHUNK)
        def _():
            issue(c0 + 2)

        wait(c0)
        acc_sv, acc_rs = process(lax.rem(c0, _NBUF), lab16, 0,
                                 acc_sv, acc_rs)

        @pl.when(c0 + 3 < _NCHUNK)
        def _():
            issue(c0 + 3)

        wait(c0 + 1)
        acc_sv, acc_rs = process(lax.rem(c0 + 1, _NBUF), lab16, _CHUNK,
                                 acc_sv, acc_rs)
        return acc_sv, acc_rs

    acc_sv, acc_rs = lax.fori_loop(0, _NSUPER, super_body, (zero, zero))

    outv[...] = acc_sv - s * acc_rs
    pltpu.sync_copy(outv,
                    out_hbm.at[pl.ds(pl.multiple_of(wid * _L, _L), _L)])


def _tc_body(lab_ref, x_ref, out_ref):
    i = pl.program_id(0)
    xb = x_ref[...]
    lab = lab_ref[0, 0, :]
    m = lab != _PADDING_IDX
    rowsum = jnp.sum(xb, axis=1)
    x0 = xb[:, 0]
    onehot = lab[:, None] == lax.broadcasted_iota(jnp.int32,
                                                  (_TC_BLK, _C), 1)
    xlab = jnp.sum(jnp.where(onehot, xb, jnp.float32(0.0)), axis=1)
    contrib = (jnp.float32(_K) - jnp.float32(_S) * (rowsum - x0)
               - jnp.float32(_CONFIDENCE - _S) * xlab)
    part = jnp.sum(jnp.where(m, contrib, jnp.float32(0.0)))

    @pl.when(i == 0)
    def _():
        out_ref[0, 0] = jnp.float32(0.0)

    out_ref[0, 0] += part


@jax.jit
def _loss(x2d, lab1d, lab3d):
    mesh = plsc.VectorSubcoreMesh(core_axis_name="c", subcore_axis_name="s")
    run = functools.partial(
        pl.kernel,
        out_type=jax.ShapeDtypeStruct((_NW * _L,), jnp.float32),
        mesh=mesh,
        scratch_types=[
            pltpu.VMEM((_NBUF, _CHUNK, _C), jnp.float32),
            pltpu.VMEM((_ROWS_PER_W,), jnp.int32),
            pltpu.VMEM((_L,), jnp.float32),
            pltpu.SemaphoreType.DMA,
        ],
        compiler_params=pltpu.CompilerParams(use_tc_tiling_on_sc=True,
                                             needs_layout_passes=False),
    )(_body)
    partials = run(x2d, lab1d)

    tc_out = pl.pallas_call(
        _tc_body,
        grid=(_TC_GRID,),
        in_specs=[
            pl.BlockSpec((1, 1, _TC_BLK), lambda i: (i, 0, 0)),
            pl.BlockSpec((_TC_BLK, _C), lambda i: (i, 0)),
        ],
        out_specs=pl.BlockSpec(memory_space=pltpu.SMEM),
        out_shape=jax.ShapeDtypeStruct((1, 1), jnp.float32),
        compiler_params=pltpu.CompilerParams(
            dimension_semantics=("arbitrary",)),
    )(lab3d, x2d)

    return jnp.sum(partials) + tc_out[0, 0]


def kernel(inputs, input_sizes, labels, label_sizes):
    x2d = inputs.reshape(_N, _C)
    lab1d = labels.reshape(_N).astype(jnp.int32)
    lab3d = lab1d[:_NT].reshape(_TC_GRID, 1, _TC_BLK)
    return _loss(x2d, lab1d, lab3d)
